# baseline (speedup 1.0000x reference)
import functools

import jax
import jax.numpy as jnp
from jax import lax
from jax.experimental import pallas as pl
from jax.experimental.pallas import tpu as pltpu
from jax.experimental.pallas import tpu_sc as plsc

_NC = 2
_NS = 16
_NW = _NC * _NS


def _make_sc_gather(d, gt, t0):
    chunks = gt // 128
    t1 = chunks // _NS - t0
    tmax = max(t0, t1)
    R = 5
    K = 3
    mesh = plsc.VectorSubcoreMesh(core_axis_name="c", subcore_axis_name="s")

    @functools.partial(
        pl.kernel,
        mesh=mesh,
        out_type=jax.ShapeDtypeStruct((gt, d), jnp.float32),
        scratch_types=(
            [pltpu.VMEM((tmax, 128), jnp.int32),
             pltpu.VMEM((R * 128, d), jnp.float32)]
            + [pltpu.SemaphoreType.DMA] * (2 * R)),
    )
    def gather_k(x_hbm, idx_hbm, out_hbm, idx_v, ring_v, *sems):
        sg, ss = sems[:R], sems[R:]
        c = lax.axis_index("c")
        s = lax.axis_index("s")
        tcnt = lax.select(c == 0, t0, t1)
        base_c = lax.select(c == 0, s * t0, _NS * t0 + s * t1)
        ng = tcnt // R

        pltpu.sync_copy(
            idx_hbm.at[pl.ds(pl.multiple_of(base_c, 8), tmax)], idx_v)
        base_w = base_c * 128

        def fire_gather(j, b):
            pltpu.async_copy(x_hbm.at[idx_v.at[j]],
                             ring_v.at[pl.ds(b * 128, 128)], sg[b])

        def wait_gather(b):
            pltpu.make_async_copy(out_hbm.at[pl.ds(0, 128)],
                                  ring_v.at[pl.ds(b * 128, 128)], sg[b]).wait()

        def fire_store(j, b):
            dst = pl.multiple_of(base_w + j * 128, 128)
            pltpu.async_copy(ring_v.at[pl.ds(b * 128, 128)],
                             out_hbm.at[pl.ds(dst, 128)], ss[b])

        def wait_store(b):
            pltpu.make_async_copy(ring_v.at[pl.ds(b * 128, 128)],
                                  out_hbm.at[pl.ds(0, 128)], ss[b]).wait()

        def group(g, first, last):
            for b in range(R):
                j = g * R + b
                wait_gather(b)
                fire_store(j, b)
                bn = (b + K) % R
                if first:
                    if b >= R - K:
                        wait_store(bn)
                    fire_gather(j + K, bn)
                elif last:
                    if b < R - K:
                        wait_store(bn)
                        fire_gather(j + K, bn)
                else:
                    wait_store(bn)
                    fire_gather(j + K, bn)

        for b in range(K):
            fire_gather(b, b)
        group(0, True, False)
        lax.fori_loop(1, ng - 1, lambda g, u: (group(g, False, False), u)[1], 0)
        group(ng - 1, False, True)
        for b in range(R):
            wait_store(b)

    return gather_k


def _make_sc_scatter(ep, d, n_half, agg_pad):
    per_w = ep // _NS
    batch = 1024
    nb = per_w // batch
    acc_rows = n_half + 1024
    init_rows = acc_rows // _NS
    drain_rows = n_half // _NS
    mesh = plsc.VectorSubcoreMesh(core_axis_name="c", subcore_axis_name="s")

    @functools.partial(
        pl.kernel,
        mesh=mesh,
        out_type=jax.ShapeDtypeStruct((agg_pad, d), jnp.float32),
        scratch_types=[
            pltpu.VMEM((8, 128), jnp.int32),
            pltpu.VMEM((512, d), jnp.float32),
            pltpu.VMEM((64, d), jnp.float32),
            pltpu.VMEM_SHARED((acc_rows, d), jnp.float32),
            pltpu.SemaphoreType.DMA,
        ],
    )
    def scatter_k(msg_hbm, ridx_hbm, zeros_hbm, out_hbm,
                  idx_v, rows_v, stage_v, acc_sh, sem):
        c = lax.axis_index("c")
        s = lax.axis_index("s")
        base_w = s * per_w
        node0 = c * n_half

        pltpu.sync_copy(zeros_hbm, stage_v)
        for t in range(init_rows // 64):
            row = pl.multiple_of(s * init_rows + t * 64, 64)
            pltpu.sync_copy(stage_v, acc_sh.at[pl.ds(row, 64)])
        plsc.subcore_barrier()

        def body(k, carry):
            base = pl.multiple_of(base_w + k * batch, batch)
            pltpu.sync_copy(ridx_hbm.at[pl.ds(pl.multiple_of(base // 128, 8), 8)],
                            idx_v)
            for r in range(8):
                for q in range(8):
                    v = idx_v[r, pl.ds(q * 16, 16)] - node0
                    ok = (v >= 0) & (v < n_half)
                    idx_v[r, pl.ds(q * 16, 16)] = jnp.where(ok, v, n_half)
            for h in range(2):
                pltpu.sync_copy(
                    msg_hbm.at[pl.ds(pl.multiple_of(base + h * 512, 512), 512)],
                    rows_v)
                for j in range(4):
                    pltpu.sync_copy(
                        rows_v.at[pl.ds(j * 128, 128)],
                        acc_sh.at[idx_v.at[4 * h + j]],
                        add=True,
                    )
            return carry

        lax.fori_loop(0, nb, body, 0)
        plsc.subcore_barrier()

        for t in range(drain_rows // 64):
            row = pl.multiple_of(s * drain_rows + t * 64, 64)
            pltpu.sync_copy(acc_sh.at[pl.ds(row, 64)], stage_v)
            pltpu.sync_copy(
                stage_v,
                out_hbm.at[pl.ds(pl.multiple_of(node0 + row, 64), 64)])

    return scatter_k


def _edge_body(xs_ref, xr_ref, ea_ref, w0a_ref, w0b_ref, w1_ref,
               b0_ref, b1_ref, o_ref):
    bf = jnp.bfloat16
    ea = ea_ref[...]
    y = jnp.dot(xs_ref[...].astype(bf), w0a_ref[...].astype(bf),
                preferred_element_type=jnp.float32)
    y = y + jnp.dot(xr_ref[...].astype(bf), w0b_ref[...].astype(bf),
                    preferred_element_type=jnp.float32)
    t = b0_ref[...]
    t = t + sum(ea[:, j:j + 1] * y[:, 128 * j:128 * (j + 1)] for j in range(4))
    m = t * jax.nn.sigmoid(t)
    y2 = jnp.dot(m.astype(bf), w1_ref[...].astype(bf),
                 preferred_element_type=jnp.float32)
    t2 = b1_ref[...]
    t2 = t2 + sum(ea[:, j:j + 1] * y2[:, 128 * j:128 * (j + 1)] for j in range(4))
    o_ref[...] = t2 * jax.nn.sigmoid(t2)


def _tc_edge(g, ea, w0a, w0b, w1, b0, b1, e_real, be):
    ep, a = ea.shape
    d = g.shape[1]
    e0 = e_real // be
    return pl.pallas_call(
        _edge_body,
        grid=(ep // be,),
        in_specs=[
            pl.BlockSpec((be, d), lambda i: (i, 0)),
            pl.BlockSpec((be, d), lambda i, e0=e0: (i + e0, 0)),
            pl.BlockSpec((be, a), lambda i: (i, 0)),
            pl.BlockSpec((d, 4 * d), lambda i: (0, 0)),
            pl.BlockSpec((d, 4 * d), lambda i: (0, 0)),
            pl.BlockSpec((d, 4 * d), lambda i: (0, 0)),
            pl.BlockSpec((1, d), lambda i: (0, 0)),
            pl.BlockSpec((1, d), lambda i: (0, 0)),
        ],
        out_specs=pl.BlockSpec((be, d), lambda i: (i, 0)),
        out_shape=jax.ShapeDtypeStruct((ep, d), jnp.float32),
        compiler_params=pltpu.CompilerParams(
            dimension_semantics=("arbitrary",)),
    )(g, g, ea, w0a, w0b, w1, b0, b1)


def _node_body(x_ref, p_ref, na_ref, wa_ref, wb_ref, wf_ref,
               b0_ref, bf_ref, o_ref):
    x = x_ref[...]
    na = na_ref[...]
    agg = p_ref[...]
    y = jnp.dot(x, wa_ref[...], preferred_element_type=jnp.float32)
    y = y + jnp.dot(agg, wb_ref[...], preferred_element_type=jnp.float32)
    t = b0_ref[...]
    t = t + sum(na[:, j:j + 1] * y[:, 128 * j:128 * (j + 1)] for j in range(4))
    h = t * jax.nn.sigmoid(t)
    y2 = jnp.dot(h, wf_ref[...], preferred_element_type=jnp.float32)
    u = bf_ref[...]
    u = u + sum(na[:, j:j + 1] * y2[:, 128 * j:128 * (j + 1)] for j in range(4))
    o_ref[...] = x + u


def _tc_node(x_pad, p, na_pad, wa, wb, wf, b0, bf, bn):
    np_, a = na_pad.shape
    d = x_pad.shape[1]
    return pl.pallas_call(
        _node_body,
        grid=(np_ // bn,),
        in_specs=[
            pl.BlockSpec((bn, d), lambda i: (i, 0)),
            pl.BlockSpec((bn, d), lambda i: (i, 0)),
            pl.BlockSpec((bn, a), lambda i: (i, 0)),
            pl.BlockSpec((d, 4 * d), lambda i: (0, 0)),
            pl.BlockSpec((d, 4 * d), lambda i: (0, 0)),
            pl.BlockSpec((d, 4 * d), lambda i: (0, 0)),
            pl.BlockSpec((1, d), lambda i: (0, 0)),
            pl.BlockSpec((1, d), lambda i: (0, 0)),
        ],
        out_specs=pl.BlockSpec((bn, d), lambda i: (i, 0)),
        out_shape=jax.ShapeDtypeStruct((np_, d), jnp.float32),
        compiler_params=pltpu.CompilerParams(
            dimension_semantics=("arbitrary",)),
    )(x_pad, p, na_pad, wa, wb, wf, b0, bf)


def kernel(x, edge_index, edge_attr, node_attr, W_msg0, b_msg0, W_msg1,
           b_msg1, W_upd0, b_upd0, W_updf, b_updf):
    n, d = x.shape
    e = edge_index.shape[1]
    a = edge_attr.shape[1]

    gt = _NW * 1024 * -(-2 * e // (_NW * 1024))
    ep = _NS * 1024 * -(-e // (_NS * 1024))
    bn = 1024
    n_pad = bn * -(-n // bn)
    n_half = n_pad // 2

    x_pad = jnp.concatenate([x, jnp.zeros((n_pad - n, d), jnp.float32)])
    idx_flat = jnp.concatenate(
        [edge_index.reshape(-1),
         jnp.zeros((gt - 2 * e,), jnp.int32)]).reshape(gt // 128, 128)
    g = _make_sc_gather(d, gt, 40)(x, idx_flat)

    ea_pad = jnp.concatenate(
        [edge_attr, jnp.zeros((ep - e, a), jnp.float32)])
    w0 = W_msg0.reshape(2 * d, a * d)
    msg = _tc_edge(g, ea_pad, w0[:d], w0[d:], W_msg1.reshape(d, a * d),
                   b_msg0.reshape(1, d), b_msg1.reshape(1, d), e, 1280)

    ridx = jnp.concatenate(
        [edge_index[1],
         jnp.full((ep - e,), n, jnp.int32)]).reshape(ep // 128, 128)
    zeros_blk = jnp.zeros((64, d), jnp.float32)
    p = _make_sc_scatter(ep, d, n_half, n_pad)(msg, ridx, zeros_blk)

    na_pad = jnp.concatenate(
        [node_attr, jnp.zeros((n_pad - n, a), jnp.float32)])
    wu = W_upd0.reshape(2 * d, a * d)
    out_pad = _tc_node(x_pad, p, na_pad, wu[:d], wu[d:],
                       W_updf.reshape(d, a * d), b_upd0.reshape(1, d),
                       b_updf.reshape(1, d), bn)
    return out_pad[:n]


# speedup vs baseline: 1.3224x; 1.3224x over previous
import functools

import jax
import jax.numpy as jnp
from jax import lax
from jax.experimental import pallas as pl
from jax.experimental.pallas import tpu as pltpu
from jax.experimental.pallas import tpu_sc as plsc

_NC = 2
_NS = 16
_NW = _NC * _NS


def _make_sc_gather(d, gt, t0):
    chunks = gt // 128
    t1 = chunks // _NS - t0
    tmax = max(t0, t1)
    R = 5
    K = 3
    mesh = plsc.VectorSubcoreMesh(core_axis_name="c", subcore_axis_name="s")

    @functools.partial(
        pl.kernel,
        mesh=mesh,
        out_type=jax.ShapeDtypeStruct((gt, d), jnp.float32),
        scratch_types=(
            [pltpu.VMEM((tmax, 128), jnp.int32),
             pltpu.VMEM((R * 128, d), jnp.float32)]
            + [pltpu.SemaphoreType.DMA] * (2 * R)),
    )
    def gather_k(x_hbm, idx_hbm, out_hbm, idx_v, ring_v, *sems):
        sg, ss = sems[:R], sems[R:]
        c = lax.axis_index("c")
        s = lax.axis_index("s")
        tcnt = lax.select(c == 0, t0, t1)
        base_c = lax.select(c == 0, s * t0, _NS * t0 + s * t1)
        ng = tcnt // R

        pltpu.sync_copy(
            idx_hbm.at[pl.ds(pl.multiple_of(base_c, 8), tmax)], idx_v)
        base_w = base_c * 128

        def fire_gather(j, b):
            pltpu.async_copy(x_hbm.at[idx_v.at[j]],
                             ring_v.at[pl.ds(b * 128, 128)], sg[b])

        def wait_gather(b):
            pltpu.make_async_copy(out_hbm.at[pl.ds(0, 128)],
                                  ring_v.at[pl.ds(b * 128, 128)], sg[b]).wait()

        def fire_store(j, b):
            dst = pl.multiple_of(base_w + j * 128, 128)
            pltpu.async_copy(ring_v.at[pl.ds(b * 128, 128)],
                             out_hbm.at[pl.ds(dst, 128)], ss[b])

        def wait_store(b):
            pltpu.make_async_copy(ring_v.at[pl.ds(b * 128, 128)],
                                  out_hbm.at[pl.ds(0, 128)], ss[b]).wait()

        def group(g, first, last):
            for b in range(R):
                j = g * R + b
                wait_gather(b)
                fire_store(j, b)
                bn = (b + K) % R
                if first:
                    if b >= R - K:
                        wait_store(bn)
                    fire_gather(j + K, bn)
                elif last:
                    if b < R - K:
                        wait_store(bn)
                        fire_gather(j + K, bn)
                else:
                    wait_store(bn)
                    fire_gather(j + K, bn)

        for b in range(K):
            fire_gather(b, b)
        group(0, True, False)
        lax.fori_loop(1, ng - 1, lambda g, u: (group(g, False, False), u)[1], 0)
        group(ng - 1, False, True)
        for b in range(R):
            wait_store(b)

    return gather_k


def _make_sc_scatter(ep, d, n_half, agg_pad):
    eh = ep // 2
    per_w = eh // _NS
    batch = 1024
    nb = per_w // batch
    acc_rows = n_half + 1024
    init_rows = acc_rows // _NS
    drain_rows = n_half // _NS
    mesh = plsc.VectorSubcoreMesh(core_axis_name="c", subcore_axis_name="s")

    @functools.partial(
        pl.kernel,
        mesh=mesh,
        out_type=jax.ShapeDtypeStruct((agg_pad, d), jnp.float32),
        scratch_types=[
            pltpu.VMEM((8, 128), jnp.int32),
            pltpu.VMEM((512, d), jnp.float32),
            pltpu.VMEM((64, d), jnp.float32),
            pltpu.VMEM_SHARED((acc_rows, d), jnp.float32),
            pltpu.SemaphoreType.DMA,
        ],
    )
    def scatter_k(msga_hbm, msgb_hbm, ridx_hbm, zeros_hbm, out_hbm,
                  idx_v, rows_v, stage_v, acc_sh, sem):
        c = lax.axis_index("c")
        s = lax.axis_index("s")
        base_w = s * per_w
        node0 = c * n_half

        pltpu.sync_copy(zeros_hbm, stage_v)
        for t in range(init_rows // 64):
            row = pl.multiple_of(s * init_rows + t * 64, 64)
            pltpu.sync_copy(stage_v, acc_sh.at[pl.ds(row, 64)])
        plsc.subcore_barrier()

        def make_body(msg_hbm, ioff):
            def body(k, carry):
                base = pl.multiple_of(base_w + k * batch, batch)
                pltpu.sync_copy(
                    ridx_hbm.at[pl.ds(pl.multiple_of(ioff + base // 128, 8), 8)],
                    idx_v)
                for r in range(8):
                    for q in range(8):
                        v = idx_v[r, pl.ds(q * 16, 16)] - node0
                        ok = (v >= 0) & (v < n_half)
                        idx_v[r, pl.ds(q * 16, 16)] = jnp.where(ok, v, n_half)
                for h in range(2):
                    pltpu.sync_copy(
                        msg_hbm.at[pl.ds(pl.multiple_of(base + h * 512, 512),
                                         512)],
                        rows_v)
                    for j in range(4):
                        pltpu.sync_copy(
                            rows_v.at[pl.ds(j * 128, 128)],
                            acc_sh.at[idx_v.at[4 * h + j]],
                            add=True,
                        )
                return carry
            return body

        lax.fori_loop(0, nb, make_body(msga_hbm, 0), 0)
        lax.fori_loop(0, nb, make_body(msgb_hbm, eh // 128), 0)
        plsc.subcore_barrier()

        for t in range(drain_rows // 64):
            row = pl.multiple_of(s * drain_rows + t * 64, 64)
            pltpu.sync_copy(acc_sh.at[pl.ds(row, 64)], stage_v)
            pltpu.sync_copy(
                stage_v,
                out_hbm.at[pl.ds(pl.multiple_of(node0 + row, 64), 64)])

    return scatter_k


def _edge_body(xs_ref, xr_ref, ea_ref, w0a_ref, w0b_ref, w1_ref,
               b0_ref, b1_ref, o_ref):
    bf = jnp.bfloat16
    ea = ea_ref[...]
    y = jnp.dot(xs_ref[...].astype(bf), w0a_ref[...].astype(bf),
                preferred_element_type=jnp.float32)
    y = y + jnp.dot(xr_ref[...].astype(bf), w0b_ref[...].astype(bf),
                    preferred_element_type=jnp.float32)
    t = b0_ref[...]
    t = t + sum(ea[:, j:j + 1] * y[:, 128 * j:128 * (j + 1)] for j in range(4))
    m = t * jax.nn.sigmoid(t)
    y2 = jnp.dot(m.astype(bf), w1_ref[...].astype(bf),
                 preferred_element_type=jnp.float32)
    t2 = b1_ref[...]
    t2 = t2 + sum(ea[:, j:j + 1] * y2[:, 128 * j:128 * (j + 1)] for j in range(4))
    o_ref[...] = t2 * jax.nn.sigmoid(t2)


def _tc_edge(g, ea, w0a, w0b, w1, b0, b1, e_real, be):
    ep, a = ea.shape
    d = g.shape[1]
    e0 = e_real // be
    return pl.pallas_call(
        _edge_body,
        grid=(ep // be,),
        in_specs=[
            pl.BlockSpec((be, d), lambda i: (i, 0)),
            pl.BlockSpec((be, d), lambda i, e0=e0: (i + e0, 0)),
            pl.BlockSpec((be, a), lambda i: (i, 0)),
            pl.BlockSpec((d, 4 * d), lambda i: (0, 0)),
            pl.BlockSpec((d, 4 * d), lambda i: (0, 0)),
            pl.BlockSpec((d, 4 * d), lambda i: (0, 0)),
            pl.BlockSpec((1, d), lambda i: (0, 0)),
            pl.BlockSpec((1, d), lambda i: (0, 0)),
        ],
        out_specs=pl.BlockSpec((be, d), lambda i: (i, 0)),
        out_shape=jax.ShapeDtypeStruct((ep, d), jnp.float32),
        compiler_params=pltpu.CompilerParams(
            dimension_semantics=("arbitrary",)),
    )(g, g, ea, w0a, w0b, w1, b0, b1)


def _node_body(x_ref, p_ref, na_ref, wa_ref, wb_ref, wf_ref,
               b0_ref, bf_ref, o_ref):
    x = x_ref[...]
    na = na_ref[...]
    agg = p_ref[...]
    y = jnp.dot(x, wa_ref[...], preferred_element_type=jnp.float32)
    y = y + jnp.dot(agg, wb_ref[...], preferred_element_type=jnp.float32)
    t = b0_ref[...]
    t = t + sum(na[:, j:j + 1] * y[:, 128 * j:128 * (j + 1)] for j in range(4))
    h = t * jax.nn.sigmoid(t)
    y2 = jnp.dot(h, wf_ref[...], preferred_element_type=jnp.float32)
    u = bf_ref[...]
    u = u + sum(na[:, j:j + 1] * y2[:, 128 * j:128 * (j + 1)] for j in range(4))
    o_ref[...] = x + u


def _tc_node(x_pad, p, na_pad, wa, wb, wf, b0, bf, bn):
    np_, a = na_pad.shape
    d = x_pad.shape[1]
    return pl.pallas_call(
        _node_body,
        grid=(np_ // bn,),
        in_specs=[
            pl.BlockSpec((bn, d), lambda i: (i, 0)),
            pl.BlockSpec((bn, d), lambda i: (i, 0)),
            pl.BlockSpec((bn, a), lambda i: (i, 0)),
            pl.BlockSpec((d, 4 * d), lambda i: (0, 0)),
            pl.BlockSpec((d, 4 * d), lambda i: (0, 0)),
            pl.BlockSpec((d, 4 * d), lambda i: (0, 0)),
            pl.BlockSpec((1, d), lambda i: (0, 0)),
            pl.BlockSpec((1, d), lambda i: (0, 0)),
        ],
        out_specs=pl.BlockSpec((bn, d), lambda i: (i, 0)),
        out_shape=jax.ShapeDtypeStruct((np_, d), jnp.float32),
        compiler_params=pltpu.CompilerParams(
            dimension_semantics=("arbitrary",)),
    )(x_pad, p, na_pad, wa, wb, wf, b0, bf)


def kernel(x, edge_index, edge_attr, node_attr, W_msg0, b_msg0, W_msg1,
           b_msg1, W_upd0, b_upd0, W_updf, b_updf):
    n, d = x.shape
    e = edge_index.shape[1]
    a = edge_attr.shape[1]

    ep = _NS * 2048 * -(-e // (_NS * 2048))
    eh = ep // 2
    bn = 1024
    n_pad = bn * -(-n // bn)
    n_half = n_pad // 2

    x_pad = jnp.concatenate([x, jnp.zeros((n_pad - n, d), jnp.float32)])
    s_pad = jnp.concatenate([edge_index[0], jnp.zeros((ep - e,), jnp.int32)])
    r_pad = jnp.concatenate([edge_index[1],
                             jnp.full((ep - e,), n, jnp.int32)])
    ea_pad = jnp.concatenate(
        [edge_attr, jnp.zeros((ep - e, a), jnp.float32)])
    w0 = W_msg0.reshape(2 * d, a * d)
    w1 = W_msg1.reshape(d, a * d)
    b0 = b_msg0.reshape(1, d)
    b1 = b_msg1.reshape(1, d)

    gather = _make_sc_gather(d, 2 * eh, (2 * eh // 128) // _NW)
    msgs = []
    for q in range(2):
        idx_q = jnp.concatenate(
            [s_pad[q * eh:(q + 1) * eh],
             r_pad[q * eh:(q + 1) * eh]]).reshape(2 * eh // 128, 128)
        g_q = gather(x, idx_q)
        msgs.append(_tc_edge(g_q, ea_pad[q * eh:(q + 1) * eh], w0[:d], w0[d:],
                             w1, b0, b1, eh, 1280))

    ridx = r_pad.reshape(ep // 128, 128)
    zeros_blk = jnp.zeros((64, d), jnp.float32)
    p = _make_sc_scatter(ep, d, n_half, n_pad)(msgs[0], msgs[1], ridx,
                                               zeros_blk)

    na_pad = jnp.concatenate(
        [node_attr, jnp.zeros((n_pad - n, a), jnp.float32)])
    wu = W_upd0.reshape(2 * d, a * d)
    out_pad = _tc_node(x_pad, p, na_pad, wu[:d], wu[d:],
                       W_updf.reshape(d, a * d), b_upd0.reshape(1, d),
                       b_updf.reshape(1, d), bn)
    return out_pad[:n]


# speedup vs baseline: 1.3367x; 1.0108x over previous
import functools

import jax
import jax.numpy as jnp
from jax import lax
from jax.experimental import pallas as pl
from jax.experimental.pallas import tpu as pltpu
from jax.experimental.pallas import tpu_sc as plsc

_NC = 2
_NS = 16
_NW = _NC * _NS


def _make_sc_gather(d, gt, q):
    tpc = gt // 128 // _NW
    hc = gt // 256
    R = 5
    K = 3
    ng = tpc // R
    mesh = plsc.VectorSubcoreMesh(core_axis_name="c", subcore_axis_name="s")

    @functools.partial(
        pl.kernel,
        mesh=mesh,
        out_type=jax.ShapeDtypeStruct((gt, d), jnp.float32),
        scratch_types=(
            [pltpu.VMEM((tpc, 128), jnp.int32),
             pltpu.VMEM((R * 128, d), jnp.float32)]
            + [pltpu.SemaphoreType.DMA] * (2 * R)),
    )
    def gather_k(x_hbm, ei_hbm, out_hbm, idx_v, ring_v, *sems):
        sg, ss = sems[:R], sems[R:]
        c = lax.axis_index("c")
        s = lax.axis_index("s")

        irow = pl.multiple_of(q * hc + s * tpc, 8)

        @pl.when(c == 0)
        def _():
            pltpu.sync_copy(ei_hbm.at[0, pl.ds(irow, tpc)], idx_v)

        @pl.when(c != 0)
        def _():
            pltpu.sync_copy(ei_hbm.at[1, pl.ds(irow, tpc)], idx_v)

        base_w = (c * hc + s * tpc) * 128

        def fire_gather(j, b):
            pltpu.async_copy(x_hbm.at[idx_v.at[j]],
                             ring_v.at[pl.ds(b * 128, 128)], sg[b])

        def wait_gather(b):
            pltpu.make_async_copy(out_hbm.at[pl.ds(0, 128)],
                                  ring_v.at[pl.ds(b * 128, 128)], sg[b]).wait()

        def fire_store(j, b):
            dst = pl.multiple_of(base_w + j * 128, 128)
            pltpu.async_copy(ring_v.at[pl.ds(b * 128, 128)],
                             out_hbm.at[pl.ds(dst, 128)], ss[b])

        def wait_store(b):
            pltpu.make_async_copy(ring_v.at[pl.ds(b * 128, 128)],
                                  out_hbm.at[pl.ds(0, 128)], ss[b]).wait()

        def group(g, first, last):
            for b in range(R):
                j = g * R + b
                wait_gather(b)
                fire_store(j, b)
                bn = (b + K) % R
                if first:
                    if b >= R - K:
                        wait_store(bn)
                    fire_gather(j + K, bn)
                elif last:
                    if b < R - K:
                        wait_store(bn)
                        fire_gather(j + K, bn)
                else:
                    wait_store(bn)
                    fire_gather(j + K, bn)

        for b in range(K):
            fire_gather(b, b)
        group(0, True, False)
        lax.fori_loop(1, ng - 1, lambda g, u: (group(g, False, False), u)[1], 0)
        group(ng - 1, False, True)
        for b in range(R):
            wait_store(b)

    return gather_k


def _make_sc_scatter(ep, d, n_half, agg_pad):
    eh = ep // 2
    per_w = eh // _NS
    batch = 1024
    nb = per_w // batch
    acc_rows = n_half + 1024
    init_rows = acc_rows // _NS
    drain_rows = n_half // _NS
    mesh = plsc.VectorSubcoreMesh(core_axis_name="c", subcore_axis_name="s")

    @functools.partial(
        pl.kernel,
        mesh=mesh,
        out_type=jax.ShapeDtypeStruct((agg_pad, d), jnp.float32),
        scratch_types=[
            pltpu.VMEM((8, 128), jnp.int32),
            pltpu.VMEM((512, d), jnp.float32),
            pltpu.VMEM((64, d), jnp.float32),
            pltpu.VMEM_SHARED((acc_rows, d), jnp.float32),
            pltpu.SemaphoreType.DMA,
        ],
    )
    def scatter_k(msga_hbm, msgb_hbm, ei_hbm, zeros_hbm, out_hbm,
                  idx_v, rows_v, stage_v, acc_sh, sem):
        c = lax.axis_index("c")
        s = lax.axis_index("s")
        base_w = s * per_w
        node0 = c * n_half

        pltpu.sync_copy(zeros_hbm, stage_v)
        for t in range(init_rows // 64):
            row = pl.multiple_of(s * init_rows + t * 64, 64)
            pltpu.sync_copy(stage_v, acc_sh.at[pl.ds(row, 64)])
        plsc.subcore_barrier()

        def make_body(msg_hbm, ioff):
            def body(k, carry):
                base = pl.multiple_of(base_w + k * batch, batch)
                pltpu.sync_copy(
                    ei_hbm.at[1, pl.ds(pl.multiple_of(ioff + base // 128, 8),
                                       8)],
                    idx_v)
                for r in range(8):
                    for q in range(8):
                        v = idx_v[r, pl.ds(q * 16, 16)] - node0
                        ok = (v >= 0) & (v < n_half)
                        idx_v[r, pl.ds(q * 16, 16)] = jnp.where(ok, v, n_half)
                for h in range(2):
                    pltpu.sync_copy(
                        msg_hbm.at[pl.ds(pl.multiple_of(base + h * 512, 512),
                                         512)],
                        rows_v)
                    for j in range(4):
                        pltpu.sync_copy(
                            rows_v.at[pl.ds(j * 128, 128)],
                            acc_sh.at[idx_v.at[4 * h + j]],
                            add=True,
                        )
                return carry
            return body

        lax.fori_loop(0, nb, make_body(msga_hbm, 0), 0)
        lax.fori_loop(0, nb, make_body(msgb_hbm, eh // 128), 0)
        plsc.subcore_barrier()

        for t in range(drain_rows // 64):
            row = pl.multiple_of(s * drain_rows + t * 64, 64)
            pltpu.sync_copy(acc_sh.at[pl.ds(row, 64)], stage_v)
            pltpu.sync_copy(
                stage_v,
                out_hbm.at[pl.ds(pl.multiple_of(node0 + row, 64), 64)])

    return scatter_k


def _edge_body(xs_ref, xr_ref, ea_ref, w0a_ref, w0b_ref, w1_ref,
               b0_ref, b1_ref, o_ref):
    bf = jnp.bfloat16
    ea = ea_ref[...]
    y = jnp.dot(xs_ref[...].astype(bf), w0a_ref[...].astype(bf),
                preferred_element_type=jnp.float32)
    y = y + jnp.dot(xr_ref[...].astype(bf), w0b_ref[...].astype(bf),
                    preferred_element_type=jnp.float32)
    t = b0_ref[...]
    t = t + sum(ea[:, j:j + 1] * y[:, 128 * j:128 * (j + 1)] for j in range(4))
    m = t * jax.nn.sigmoid(t)
    y2 = jnp.dot(m.astype(bf), w1_ref[...].astype(bf),
                 preferred_element_type=jnp.float32)
    t2 = b1_ref[...]
    t2 = t2 + sum(ea[:, j:j + 1] * y2[:, 128 * j:128 * (j + 1)] for j in range(4))
    o_ref[...] = t2 * jax.nn.sigmoid(t2)


def _tc_edge(g, ea, w0a, w0b, w1, b0, b1, eh, be, qo):
    a = ea.shape[1]
    d = g.shape[1]
    e0 = eh // be
    return pl.pallas_call(
        _edge_body,
        grid=(eh // be,),
        in_specs=[
            pl.BlockSpec((be, d), lambda i: (i, 0)),
            pl.BlockSpec((be, d), lambda i, e0=e0: (i + e0, 0)),
            pl.BlockSpec((be, a), lambda i, qo=qo: (i + qo, 0)),
            pl.BlockSpec((d, 4 * d), lambda i: (0, 0)),
            pl.BlockSpec((d, 4 * d), lambda i: (0, 0)),
            pl.BlockSpec((d, 4 * d), lambda i: (0, 0)),
            pl.BlockSpec((1, d), lambda i: (0, 0)),
            pl.BlockSpec((1, d), lambda i: (0, 0)),
        ],
        out_specs=pl.BlockSpec((be, d), lambda i: (i, 0)),
        out_shape=jax.ShapeDtypeStruct((eh, d), jnp.float32),
        compiler_params=pltpu.CompilerParams(
            dimension_semantics=("arbitrary",)),
    )(g, g, ea, w0a, w0b, w1, b0, b1)


def _node_body(x_ref, p_ref, na_ref, wa_ref, wb_ref, wf_ref,
               b0_ref, bf_ref, o_ref):
    x = x_ref[...]
    na = na_ref[...]
    agg = p_ref[...]
    y = jnp.dot(x, wa_ref[...], preferred_element_type=jnp.float32)
    y = y + jnp.dot(agg, wb_ref[...], preferred_element_type=jnp.float32)
    t = b0_ref[...]
    t = t + sum(na[:, j:j + 1] * y[:, 128 * j:128 * (j + 1)] for j in range(4))
    h = t * jax.nn.sigmoid(t)
    y2 = jnp.dot(h, wf_ref[...], preferred_element_type=jnp.float32)
    u = bf_ref[...]
    u = u + sum(na[:, j:j + 1] * y2[:, 128 * j:128 * (j + 1)] for j in range(4))
    o_ref[...] = x + u


def _tc_node(x_pad, p, na_pad, wa, wb, wf, b0, bf, bn):
    np_, a = na_pad.shape
    d = x_pad.shape[1]
    return pl.pallas_call(
        _node_body,
        grid=(np_ // bn,),
        in_specs=[
            pl.BlockSpec((bn, d), lambda i: (i, 0)),
            pl.BlockSpec((bn, d), lambda i: (i, 0)),
            pl.BlockSpec((bn, a), lambda i: (i, 0)),
            pl.BlockSpec((d, 4 * d), lambda i: (0, 0)),
            pl.BlockSpec((d, 4 * d), lambda i: (0, 0)),
            pl.BlockSpec((d, 4 * d), lambda i: (0, 0)),
            pl.BlockSpec((1, d), lambda i: (0, 0)),
            pl.BlockSpec((1, d), lambda i: (0, 0)),
        ],
        out_specs=pl.BlockSpec((bn, d), lambda i: (i, 0)),
        out_shape=jax.ShapeDtypeStruct((np_, d), jnp.float32),
        compiler_params=pltpu.CompilerParams(
            dimension_semantics=("arbitrary",)),
    )(x_pad, p, na_pad, wa, wb, wf, b0, bf)


def kernel(x, edge_index, edge_attr, node_attr, W_msg0, b_msg0, W_msg1,
           b_msg1, W_upd0, b_upd0, W_updf, b_updf):
    n, d = x.shape
    e = edge_index.shape[1]
    a = edge_attr.shape[1]

    ep = _NS * 2048 * -(-e // (_NS * 2048))
    eh = ep // 2
    bn = 1024
    n_pad = bn * -(-n // bn)
    n_half = n_pad // 2

    ei_pad = jnp.concatenate(
        [edge_index,
         jnp.concatenate([jnp.zeros((1, ep - e), jnp.int32),
                          jnp.full((1, ep - e), n, jnp.int32)])],
        axis=1).reshape(2, ep // 128, 128)
    ea_pad = jnp.concatenate(
        [edge_attr, jnp.zeros((ep - e, a), jnp.float32)])
    w0 = W_msg0.reshape(2 * d, a * d)
    w1 = W_msg1.reshape(d, a * d)
    b0 = b_msg0.reshape(1, d)
    b1 = b_msg1.reshape(1, d)

    msgs = []
    for q in range(2):
        g_q = _make_sc_gather(d, 2 * eh, q)(x, ei_pad)
        msgs.append(_tc_edge(g_q, ea_pad, w0[:d], w0[d:],
                             w1, b0, b1, eh, 1280, q * (eh // 1280)))

    zeros_blk = jnp.zeros((64, d), jnp.float32)
    p = _make_sc_scatter(ep, d, n_half, n_pad)(msgs[0], msgs[1], ei_pad,
                                               zeros_blk)

    wu = W_upd0.reshape(2 * d, a * d)
    return _tc_node(x, p, node_attr, wu[:d], wu[d:],
                    W_updf.reshape(d, a * d), b_upd0.reshape(1, d),
                    b_updf.reshape(1, d), 1000)


# speedup vs baseline: 1.4113x; 1.0558x over previous
import functools

import jax
import jax.numpy as jnp
from jax import lax
from jax.experimental import pallas as pl
from jax.experimental.pallas import tpu as pltpu
from jax.experimental.pallas import tpu_sc as plsc

_NC = 2
_NS = 16
_NW = _NC * _NS


def _make_sc_gather(d, gt, q):
    tpc = gt // 128 // _NW
    hc = gt // 256
    R = 5
    K = 3
    ng = tpc // R
    mesh = plsc.VectorSubcoreMesh(core_axis_name="c", subcore_axis_name="s")

    @functools.partial(
        pl.kernel,
        mesh=mesh,
        out_type=jax.ShapeDtypeStruct((gt, d), jnp.float32),
        scratch_types=(
            [pltpu.VMEM((tpc, 128), jnp.int32),
             pltpu.VMEM((R * 128, d), jnp.float32)]
            + [pltpu.SemaphoreType.DMA] * (2 * R)),
    )
    def gather_k(x_hbm, ei_hbm, out_hbm, idx_v, ring_v, *sems):
        sg, ss = sems[:R], sems[R:]
        c = lax.axis_index("c")
        s = lax.axis_index("s")

        irow = pl.multiple_of(q * hc + s * tpc, 8)

        @pl.when(c == 0)
        def _():
            pltpu.sync_copy(ei_hbm.at[0, pl.ds(irow, tpc)], idx_v)

        @pl.when(c != 0)
        def _():
            pltpu.sync_copy(ei_hbm.at[1, pl.ds(irow, tpc)], idx_v)

        base_w = (c * hc + s * tpc) * 128

        def fire_gather(j, b):
            pltpu.async_copy(x_hbm.at[idx_v.at[j]],
                             ring_v.at[pl.ds(b * 128, 128)], sg[b])

        def wait_gather(b):
            pltpu.make_async_copy(out_hbm.at[pl.ds(0, 128)],
                                  ring_v.at[pl.ds(b * 128, 128)], sg[b]).wait()

        def fire_store(j, b):
            dst = pl.multiple_of(base_w + j * 128, 128)
            pltpu.async_copy(ring_v.at[pl.ds(b * 128, 128)],
                             out_hbm.at[pl.ds(dst, 128)], ss[b])

        def wait_store(b):
            pltpu.make_async_copy(ring_v.at[pl.ds(b * 128, 128)],
                                  out_hbm.at[pl.ds(0, 128)], ss[b]).wait()

        def group(g, first, last):
            for b in range(R):
                j = g * R + b
                wait_gather(b)
                fire_store(j, b)
                bn = (b + K) % R
                if first:
                    if b >= R - K:
                        wait_store(bn)
                    fire_gather(j + K, bn)
                elif last:
                    if b < R - K:
                        wait_store(bn)
                        fire_gather(j + K, bn)
                else:
                    wait_store(bn)
                    fire_gather(j + K, bn)

        for b in range(K):
            fire_gather(b, b)
        group(0, True, False)
        lax.fori_loop(1, ng - 1, lambda g, u: (group(g, False, False), u)[1], 0)
        group(ng - 1, False, True)
        for b in range(R):
            wait_store(b)

    return gather_k


def _make_sc_scatter(ep, d, n_half, agg_pad):
    eh = ep // 2
    per_w = eh // _NS
    batch = 1024
    nb = per_w // batch
    acc_rows = n_half + 1024
    init_rows = acc_rows // _NS
    drain_rows = n_half // _NS
    mesh = plsc.VectorSubcoreMesh(core_axis_name="c", subcore_axis_name="s")

    @functools.partial(
        pl.kernel,
        mesh=mesh,
        out_type=jax.ShapeDtypeStruct((agg_pad, d), jnp.float32),
        scratch_types=[
            pltpu.VMEM((8, 128), jnp.int32),
            pltpu.VMEM((512, d), jnp.float32),
            pltpu.VMEM((64, d), jnp.float32),
            pltpu.VMEM_SHARED((acc_rows, d), jnp.float32),
            pltpu.SemaphoreType.DMA,
        ],
    )
    def scatter_k(msga_hbm, msgb_hbm, ei_hbm, zeros_hbm, out_hbm,
                  idx_v, rows_v, stage_v, acc_sh, sem):
        c = lax.axis_index("c")
        s = lax.axis_index("s")
        base_w = s * per_w
        node0 = c * n_half

        pltpu.sync_copy(zeros_hbm, stage_v)
        for t in range(init_rows // 64):
            row = pl.multiple_of(s * init_rows + t * 64, 64)
            pltpu.sync_copy(stage_v, acc_sh.at[pl.ds(row, 64)])
        plsc.subcore_barrier()

        def make_body(msg_hbm, ioff):
            def body(k, carry):
                base = pl.multiple_of(base_w + k * batch, batch)
                pltpu.sync_copy(
                    ei_hbm.at[1, pl.ds(pl.multiple_of(ioff + base // 128, 8),
                                       8)],
                    idx_v)
                for r in range(8):
                    for q in range(8):
                        v = idx_v[r, pl.ds(q * 16, 16)] - node0
                        ok = (v >= 0) & (v < n_half)
                        idx_v[r, pl.ds(q * 16, 16)] = jnp.where(ok, v, n_half)
                for h in range(2):
                    pltpu.sync_copy(
                        msg_hbm.at[pl.ds(pl.multiple_of(base + h * 512, 512),
                                         512)],
                        rows_v)
                    for j in range(4):
                        pltpu.sync_copy(
                            rows_v.at[pl.ds(j * 128, 128)],
                            acc_sh.at[idx_v.at[4 * h + j]],
                            add=True,
                        )
                return carry
            return body

        lax.fori_loop(0, nb, make_body(msga_hbm, 0), 0)
        lax.fori_loop(0, nb, make_body(msgb_hbm, eh // 128), 0)
        plsc.subcore_barrier()

        for t in range(drain_rows // 64):
            row = pl.multiple_of(s * drain_rows + t * 64, 64)
            pltpu.sync_copy(acc_sh.at[pl.ds(row, 64)], stage_v)
            pltpu.sync_copy(
                stage_v,
                out_hbm.at[pl.ds(pl.multiple_of(node0 + row, 64), 64)])

    return scatter_k


def _edge_body(xs_ref, xr_ref, ea_ref, w0a_ref, w0b_ref, w1_ref,
               b0_ref, b1_ref, o_ref):
    bf = jnp.bfloat16
    ea = ea_ref[...]
    y = jnp.dot(xs_ref[...].astype(bf), w0a_ref[...].astype(bf),
                preferred_element_type=jnp.float32)
    y = y + jnp.dot(xr_ref[...].astype(bf), w0b_ref[...].astype(bf),
                    preferred_element_type=jnp.float32)
    t = b0_ref[...]
    t = t + sum(ea[:, j:j + 1] * y[:, 128 * j:128 * (j + 1)] for j in range(4))
    m = t * jax.nn.sigmoid(t)
    y2 = jnp.dot(m.astype(bf), w1_ref[...].astype(bf),
                 preferred_element_type=jnp.float32)
    t2 = b1_ref[...]
    t2 = t2 + sum(ea[:, j:j + 1] * y2[:, 128 * j:128 * (j + 1)] for j in range(4))
    o_ref[...] = t2 * jax.nn.sigmoid(t2)


def _tc_edge(g, ea, w0a, w0b, w1, b0, b1, eh, be, qo):
    a = ea.shape[1]
    d = g.shape[1]
    e0 = eh // be
    emax = ea.shape[0] // be - 1
    return pl.pallas_call(
        _edge_body,
        grid=(eh // be,),
        in_specs=[
            pl.BlockSpec((be, d), lambda i: (i, 0)),
            pl.BlockSpec((be, d), lambda i, e0=e0: (i + e0, 0)),
            pl.BlockSpec((be, a),
                         lambda i, qo=qo, emax=emax: (
                             jnp.minimum(i + qo, emax), 0)),
            pl.BlockSpec((d, 4 * d), lambda i: (0, 0)),
            pl.BlockSpec((d, 4 * d), lambda i: (0, 0)),
            pl.BlockSpec((d, 4 * d), lambda i: (0, 0)),
            pl.BlockSpec((1, d), lambda i: (0, 0)),
            pl.BlockSpec((1, d), lambda i: (0, 0)),
        ],
        out_specs=pl.BlockSpec((be, d), lambda i: (i, 0)),
        out_shape=jax.ShapeDtypeStruct((eh, d), jnp.float32),
        compiler_params=pltpu.CompilerParams(
            dimension_semantics=("arbitrary",)),
    )(g, g, ea, w0a, w0b, w1, b0, b1)


def _node_body(x_ref, p_ref, na_ref, wa_ref, wb_ref, wf_ref,
               b0_ref, bf_ref, o_ref):
    x = x_ref[...]
    na = na_ref[...]
    agg = p_ref[...]
    y = jnp.dot(x, wa_ref[...], preferred_element_type=jnp.float32)
    y = y + jnp.dot(agg, wb_ref[...], preferred_element_type=jnp.float32)
    t = b0_ref[...]
    t = t + sum(na[:, j:j + 1] * y[:, 128 * j:128 * (j + 1)] for j in range(4))
    h = t * jax.nn.sigmoid(t)
    y2 = jnp.dot(h, wf_ref[...], preferred_element_type=jnp.float32)
    u = bf_ref[...]
    u = u + sum(na[:, j:j + 1] * y2[:, 128 * j:128 * (j + 1)] for j in range(4))
    o_ref[...] = x + u


def _tc_node(x_pad, p, na_pad, wa, wb, wf, b0, bf, bn):
    np_, a = na_pad.shape
    d = x_pad.shape[1]
    return pl.pallas_call(
        _node_body,
        grid=(np_ // bn,),
        in_specs=[
            pl.BlockSpec((bn, d), lambda i: (i, 0)),
            pl.BlockSpec((bn, d), lambda i: (i, 0)),
            pl.BlockSpec((bn, a), lambda i: (i, 0)),
            pl.BlockSpec((d, 4 * d), lambda i: (0, 0)),
            pl.BlockSpec((d, 4 * d), lambda i: (0, 0)),
            pl.BlockSpec((d, 4 * d), lambda i: (0, 0)),
            pl.BlockSpec((1, d), lambda i: (0, 0)),
            pl.BlockSpec((1, d), lambda i: (0, 0)),
        ],
        out_specs=pl.BlockSpec((bn, d), lambda i: (i, 0)),
        out_shape=jax.ShapeDtypeStruct((np_, d), jnp.float32),
        compiler_params=pltpu.CompilerParams(
            dimension_semantics=("arbitrary",)),
    )(x_pad, p, na_pad, wa, wb, wf, b0, bf)


def kernel(x, edge_index, edge_attr, node_attr, W_msg0, b_msg0, W_msg1,
           b_msg1, W_upd0, b_upd0, W_updf, b_updf):
    n, d = x.shape
    e = edge_index.shape[1]
    a = edge_attr.shape[1]

    ep = _NS * 2048 * -(-e // (_NS * 2048))
    eh = ep // 2
    bn = 1024
    n_pad = bn * -(-n // bn)
    n_half = n_pad // 2

    ei_pad = jnp.concatenate(
        [edge_index,
         jnp.concatenate([jnp.zeros((1, ep - e), jnp.int32),
                          jnp.full((1, ep - e), n, jnp.int32)])],
        axis=1).reshape(2, ep // 128, 128)
    w0 = W_msg0.reshape(2 * d, a * d)
    w1 = W_msg1.reshape(d, a * d)
    b0 = b_msg0.reshape(1, d)
    b1 = b_msg1.reshape(1, d)

    msgs = []
    for q in range(2):
        g_q = _make_sc_gather(d, 2 * eh, q)(x, ei_pad)
        msgs.append(_tc_edge(g_q, edge_attr, w0[:d], w0[d:],
                             w1, b0, b1, eh, 1280, q * (eh // 1280)))

    zeros_blk = jnp.zeros((64, d), jnp.float32)
    p = _make_sc_scatter(ep, d, n_half, n_pad)(msgs[0], msgs[1], ei_pad,
                                               zeros_blk)

    wu = W_upd0.reshape(2 * d, a * d)
    return _tc_node(x, p, node_attr, wu[:d], wu[d:],
                    W_updf.reshape(d, a * d), b_upd0.reshape(1, d),
                    b_updf.reshape(1, d), 1000)
